# baseline (speedup 1.0000x reference)
import jax
import jax.numpy as jnp
from jax import lax
from jax.experimental import pallas as pl
from jax.experimental.pallas import tpu as pltpu
from jax.experimental.pallas import tpu_sc as plsc

NUM_CORES = 2
NUM_SUBCORES = 16
NW = NUM_CORES * NUM_SUBCORES
B = 16384
D = 32
BPW = B // NW
CHUNKS = BPW // 16


def _mf_kernel(uids_hbm, iids_hbm, ut_hbm, it_hbm, wb_hbm, out_hbm,
               idx_u, idx_i, u_bf, i_bf, wb_v, out_v, sem_u, sem_i):
    wid = lax.axis_index("s") * NUM_CORES + lax.axis_index("c")
    base = wid * BPW

    pltpu.sync_copy(wb_hbm, wb_v)
    pltpu.sync_copy(uids_hbm.at[pl.ds(base, BPW)], idx_u)
    pltpu.sync_copy(iids_hbm.at[pl.ds(base, BPW)], idx_i)

    cu = pltpu.async_copy(ut_hbm.at[idx_u], u_bf, sem_u)
    ci = pltpu.async_copy(it_hbm.at[idx_i], i_bf, sem_i)
    cu.wait()
    ci.wait()

    w_even = wb_v[pl.ds(0, 16)]
    w_odd = wb_v[pl.ds(16, 16)]
    bias = wb_v[pl.ds(24, 16)][8]
    lanes = lax.iota(jnp.int32, 16)

    def body(c, carry):
        acc = jnp.full((16,), 0.0, dtype=jnp.float32)
        for j in range(16):
            row = c * 16 + j
            u_row = u_bf[row, :]
            i_row = i_bf[row, :]
            ue, uo = plsc.unpack(u_row, format=plsc.PackFormat.INTERLEAVED)
            ie, io = plsc.unpack(i_row, format=plsc.PackFormat.INTERLEAVED)
            t = ue * ie * w_even + uo * io * w_odd
            s = jnp.sum(t, axis=0)
            acc = jnp.where(lanes == j, s, acc)
        out_v[pl.ds(c * 16, 16)] = 1.0 / (1.0 + jnp.exp(-(acc + bias)))
        return carry

    lax.fori_loop(0, CHUNKS, body, 0)
    pltpu.sync_copy(out_v, out_hbm.at[pl.ds(base, BPW)])


@jax.jit
def kernel(input, user_table, item_table, W, b):
    uids = input[:, 0]
    iids = input[:, 1]
    ut_bf = user_table.astype(jnp.bfloat16)
    it_bf = item_table.astype(jnp.bfloat16)
    w = W.reshape(-1)
    wb = jnp.concatenate(
        [w[0::2], w[1::2], b.reshape(-1), jnp.zeros((7,), jnp.float32)])

    mesh = plsc.VectorSubcoreMesh(core_axis_name="c", subcore_axis_name="s")
    run = pl.kernel(
        _mf_kernel,
        mesh=mesh,
        compiler_params=pltpu.CompilerParams(
            needs_layout_passes=False, use_tc_tiling_on_sc=False),
        out_type=jax.ShapeDtypeStruct((B,), jnp.float32),
        scratch_types=[
            pltpu.VMEM((BPW,), jnp.int32),
            pltpu.VMEM((BPW,), jnp.int32),
            pltpu.VMEM((BPW, D), jnp.bfloat16),
            pltpu.VMEM((BPW, D), jnp.bfloat16),
            pltpu.VMEM((D + 8,), jnp.float32),
            pltpu.VMEM((BPW,), jnp.float32),
            pltpu.SemaphoreType.DMA,
            pltpu.SemaphoreType.DMA,
        ],
    )
    return run(uids, iids, ut_bf, it_bf, wb)


# speedup vs baseline: 1.1468x; 1.1468x over previous
import jax
import jax.numpy as jnp
from jax import lax
from jax.experimental import pallas as pl
from jax.experimental.pallas import tpu as pltpu
from jax.experimental.pallas import tpu_sc as plsc

NUM_CORES = 2
NUM_SUBCORES = 16
NW = NUM_CORES * NUM_SUBCORES
B = 16384
D = 32
BPW = B // NW
CHUNKS = BPW // 16


def _mf_kernel(uids_hbm, iids_hbm, ut_hbm, it_hbm, wb_hbm, out_hbm,
               idx_u, idx_i, u_rows, i_rows, wb_v, out_v, sem_u, sem_i):
    wid = lax.axis_index("s") * NUM_CORES + lax.axis_index("c")
    base = wid * BPW

    pltpu.sync_copy(wb_hbm, wb_v)
    pltpu.sync_copy(uids_hbm.at[pl.ds(base, BPW)], idx_u)
    pltpu.sync_copy(iids_hbm.at[pl.ds(base, BPW)], idx_i)

    cu = pltpu.async_copy(ut_hbm.at[idx_u], u_rows, sem_u)
    ci = pltpu.async_copy(it_hbm.at[idx_i], i_rows, sem_i)
    cu.wait()
    ci.wait()

    w_lo = wb_v[pl.ds(0, 16)]
    w_hi = wb_v[pl.ds(16, 16)]
    bias = wb_v[pl.ds(24, 16)][8]

    def body(c, carry):
        rows = c * 16 + lax.iota(jnp.int32, 16)
        acc = jnp.full((16,), 0.0, dtype=jnp.float32) + bias
        for d in range(D):
            col = jnp.full((16,), d, dtype=jnp.int32)
            ucol = plsc.load_gather(u_rows, [rows, col])
            icol = plsc.load_gather(i_rows, [rows, col])
            w_d = w_lo[d] if d < 16 else w_hi[d - 16]
            acc = acc + ucol * icol * w_d
        z = 1.0 / (1.0 + jnp.exp(-acc))
        out_v[pl.ds(c * 16, 16)] = z
        return carry

    lax.fori_loop(0, CHUNKS, body, 0)
    pltpu.sync_copy(out_v, out_hbm.at[pl.ds(base, BPW)])


@jax.jit
def kernel(input, user_table, item_table, W, b):
    uids = input[:, 0]
    iids = input[:, 1]
    wb = jnp.concatenate(
        [W.reshape(-1), b.reshape(-1), jnp.zeros((7,), jnp.float32)])

    mesh = plsc.VectorSubcoreMesh(core_axis_name="c", subcore_axis_name="s")
    run = pl.kernel(
        _mf_kernel,
        mesh=mesh,
        compiler_params=pltpu.CompilerParams(
            needs_layout_passes=False, use_tc_tiling_on_sc=False),
        out_type=jax.ShapeDtypeStruct((B,), jnp.float32),
        scratch_types=[
            pltpu.VMEM((BPW,), jnp.int32),
            pltpu.VMEM((BPW,), jnp.int32),
            pltpu.VMEM((BPW, D), jnp.float32),
            pltpu.VMEM((BPW, D), jnp.float32),
            pltpu.VMEM((D + 8,), jnp.float32),
            pltpu.VMEM((BPW,), jnp.float32),
            pltpu.SemaphoreType.DMA,
            pltpu.SemaphoreType.DMA,
        ],
    )
    return run(uids, iids, user_table, item_table, wb)
